# Initial kernel scaffold; baseline (speedup 1.0000x reference)
import jax
import jax.numpy as jnp
from jax.experimental import pallas as pl


def kernel(texts, table, W1, b1, W2, b2):
    raise NotImplementedError("write your pallas kernel here")


# speedup vs baseline: 10.6690x; 10.6690x over previous
import functools

import jax
import jax.numpy as jnp
from jax import lax
from jax.experimental import pallas as pl
from jax.experimental.pallas import tpu as pltpu
from jax.experimental.pallas import tpu_sc as plsc

_LANES = 16


@functools.partial(jax.jit, static_argnames=("nchunk", "k"))
def _pool(idx3, table, *, nchunk, k):
    B = idx3.shape[0]
    L = nchunk * k
    E = table.shape[1]

    info = plsc.get_sparse_core_info()
    nw = info.num_cores * info.num_subcores
    bpw = B // nw
    ncol = E // _LANES
    runroll = 4

    mesh = plsc.VectorSubcoreMesh(core_axis_name="c", subcore_axis_name="s")

    @functools.partial(
        pl.kernel,
        out_type=jax.ShapeDtypeStruct((B, E), jnp.float32),
        mesh=mesh,
        scratch_types=[
            pltpu.VMEM((bpw, nchunk, k), jnp.int32),
            pltpu.VMEM((L, E), jnp.float32),
            pltpu.VMEM((L, E), jnp.float32),
            pltpu.VMEM((bpw, E), jnp.float32),
            pltpu.SemaphoreType.DMA,
            pltpu.SemaphoreType.DMA,
        ],
        compiler_params=pltpu.CompilerParams(use_tc_tiling_on_sc=False),
    )
    def pool(texts_hbm, table_hbm, out_hbm, idx_v, rows0, rows1, pooled_v,
             sem0, sem1):
        wid = lax.axis_index("s") * info.num_cores + lax.axis_index("c")
        base = wid * bpw

        pltpu.sync_copy(texts_hbm.at[pl.ds(base, bpw)], idx_v)

        def gather_bag(b, buf, sem):
            for j in range(nchunk):
                pltpu.async_copy(
                    table_hbm.at[idx_v.at[b, j]],
                    buf.at[pl.ds(j * k, k)],
                    sem,
                )

        def drain_bag(b, buf, sem):
            for j in range(nchunk):
                pltpu.make_async_copy(
                    table_hbm.at[idx_v.at[b, j]],
                    buf.at[pl.ds(j * k, k)],
                    sem,
                ).wait()

        inv_l = jnp.float32(1.0 / L)

        def reduce_bag(b, buf):
            def rbody(r, accs):
                out = list(accs)
                for u in range(runroll):
                    row = r * runroll + u
                    for c in range(ncol):
                        out[c] = out[c] + buf[row, pl.ds(c * _LANES, _LANES)]
                return tuple(out)

            zero = jnp.zeros((_LANES,), jnp.float32)
            accs = lax.fori_loop(0, L // runroll, rbody, (zero,) * ncol)
            for c in range(ncol):
                pooled_v[b, pl.ds(c * _LANES, _LANES)] = accs[c] * inv_l

        gather_bag(0, rows0, sem0)

        def body(i, carry):
            b = 2 * i
            gather_bag(b + 1, rows1, sem1)
            drain_bag(b, rows0, sem0)
            reduce_bag(b, rows0)

            @pl.when(b + 2 < bpw)
            def _():
                gather_bag(b + 2, rows0, sem0)

            drain_bag(b + 1, rows1, sem1)
            reduce_bag(b + 1, rows1)
            return carry

        lax.fori_loop(0, bpw // 2, body, 0)

        pltpu.sync_copy(pooled_v, out_hbm.at[pl.ds(base, bpw)])

    return pool(idx3, table)


def _mlp(pooled, W1, b1, W2, b2):
    B, E = pooled.shape
    H = W1.shape[1]
    C = W2.shape[1]
    bb = 512
    cb = 512
    grid = (B // bb, pl.cdiv(C, cb))

    def body(p_ref, w1_ref, b1_ref, w2_ref, b2_ref, o_ref):
        h = jnp.dot(p_ref[...], w1_ref[...],
                    preferred_element_type=jnp.float32) + b1_ref[...]
        h = jnp.maximum(h, 0.0)
        o_ref[...] = jnp.dot(h, w2_ref[...],
                             preferred_element_type=jnp.float32) + b2_ref[...]

    return pl.pallas_call(
        body,
        grid=grid,
        in_specs=[
            pl.BlockSpec((bb, E), lambda i, j: (i, 0)),
            pl.BlockSpec((E, H), lambda i, j: (0, 0)),
            pl.BlockSpec((1, H), lambda i, j: (0, 0)),
            pl.BlockSpec((H, cb), lambda i, j: (0, j)),
            pl.BlockSpec((1, cb), lambda i, j: (0, j)),
        ],
        out_specs=pl.BlockSpec((bb, cb), lambda i, j: (i, j)),
        out_shape=jax.ShapeDtypeStruct((B, C), jnp.float32),
    )(pooled, W1, b1.reshape(1, H), W2, b2.reshape(1, C))


def kernel(texts, table, W1, b1, W2, b2):
    B, L = texts.shape
    nchunk = 2
    k = L // nchunk
    idx3 = texts.astype(jnp.int32).reshape(B, nchunk, k)
    pooled = _pool(idx3, table, nchunk=nchunk, k=k)
    return _mlp(pooled, W1, b1, W2, b2)


# speedup vs baseline: 12.4771x; 1.1695x over previous
import functools

import jax
import jax.numpy as jnp
from jax import lax
from jax.experimental import pallas as pl
from jax.experimental.pallas import tpu as pltpu
from jax.experimental.pallas import tpu_sc as plsc

_LANES = 16


@functools.partial(jax.jit, static_argnames=("nchunk", "k"))
def _pool(idx3, table, *, nchunk, k):
    B = idx3.shape[0]
    L = nchunk * k
    E = table.shape[1]

    info = plsc.get_sparse_core_info()
    nw = info.num_cores * info.num_subcores
    bpw = B // nw
    ncol = E // _LANES
    runroll = 4

    mesh = plsc.VectorSubcoreMesh(core_axis_name="c", subcore_axis_name="s")

    @functools.partial(
        pl.kernel,
        out_type=jax.ShapeDtypeStruct((B, E), jnp.float32),
        mesh=mesh,
        scratch_types=[
            pltpu.VMEM((bpw, nchunk * k), jnp.int32),
            pltpu.VMEM((L, E), jnp.float32),
            pltpu.VMEM((L, E), jnp.float32),
            pltpu.VMEM((bpw, E), jnp.float32),
            pltpu.SemaphoreType.DMA,
            pltpu.SemaphoreType.DMA,
        ],
        compiler_params=pltpu.CompilerParams(use_tc_tiling_on_sc=False),
    )
    def pool(texts_hbm, table_hbm, out_hbm, idx_v, rows0, rows1, pooled_v,
             sem0, sem1):
        wid = lax.axis_index("s") * info.num_cores + lax.axis_index("c")
        base = wid * bpw

        pltpu.sync_copy(texts_hbm.at[pl.ds(base, bpw)], idx_v)

        def gather_bag(b, buf, sem):
            for j in range(nchunk):
                pltpu.async_copy(
                    table_hbm.at[idx_v.at[b, pl.ds(j * k, k)]],
                    buf.at[pl.ds(j * k, k)],
                    sem,
                )

        def drain_bag(b, buf, sem):
            for j in range(nchunk):
                pltpu.make_async_copy(
                    table_hbm.at[idx_v.at[b, pl.ds(j * k, k)]],
                    buf.at[pl.ds(j * k, k)],
                    sem,
                ).wait()

        inv_l = jnp.float32(1.0 / L)

        def reduce_bag(b, buf):
            def rbody(r, accs):
                out = list(accs)
                for u in range(runroll):
                    row = r * runroll + u
                    for c in range(ncol):
                        out[c] = out[c] + buf[row, pl.ds(c * _LANES, _LANES)]
                return tuple(out)

            zero = jnp.zeros((_LANES,), jnp.float32)
            accs = lax.fori_loop(0, L // runroll, rbody, (zero,) * ncol)
            for c in range(ncol):
                pooled_v[b, pl.ds(c * _LANES, _LANES)] = accs[c] * inv_l

        gather_bag(0, rows0, sem0)

        def body(i, carry):
            b = 2 * i
            gather_bag(b + 1, rows1, sem1)
            drain_bag(b, rows0, sem0)
            reduce_bag(b, rows0)

            @pl.when(b + 2 < bpw)
            def _():
                gather_bag(b + 2, rows0, sem0)

            drain_bag(b + 1, rows1, sem1)
            reduce_bag(b + 1, rows1)
            return carry

        lax.fori_loop(0, bpw // 2, body, 0)

        pltpu.sync_copy(pooled_v, out_hbm.at[pl.ds(base, bpw)])

    return pool(idx3, table)


def _mlp(pooled, W1, b1, W2, b2):
    B, E = pooled.shape
    H = W1.shape[1]
    C = W2.shape[1]
    bb = 512
    cb = C
    grid = (B // bb, pl.cdiv(C, cb))

    def body(p_ref, w1_ref, b1_ref, w2_ref, b2_ref, o_ref):
        h = jnp.dot(p_ref[...], w1_ref[...],
                    preferred_element_type=jnp.float32) + b1_ref[...]
        h = jnp.maximum(h, 0.0)
        o_ref[...] = jnp.dot(h, w2_ref[...],
                             preferred_element_type=jnp.float32) + b2_ref[...]

    return pl.pallas_call(
        body,
        grid=grid,
        in_specs=[
            pl.BlockSpec((bb, E), lambda i, j: (i, 0)),
            pl.BlockSpec((E, H), lambda i, j: (0, 0)),
            pl.BlockSpec((1, H), lambda i, j: (0, 0)),
            pl.BlockSpec((H, cb), lambda i, j: (0, j)),
            pl.BlockSpec((1, cb), lambda i, j: (0, j)),
        ],
        out_specs=pl.BlockSpec((bb, cb), lambda i, j: (i, j)),
        out_shape=jax.ShapeDtypeStruct((B, C), jnp.float32),
    )(pooled, W1, b1.reshape(1, H), W2, b2.reshape(1, C))


def kernel(texts, table, W1, b1, W2, b2):
    B, L = texts.shape
    nchunk = 5
    k = L // nchunk
    pooled = _pool(texts.astype(jnp.int32), table, nchunk=nchunk, k=k)
    return _mlp(pooled, W1, b1, W2, b2)


# speedup vs baseline: 13.8892x; 1.1132x over previous
import functools

import jax
import jax.numpy as jnp
from jax import lax
from jax.experimental import pallas as pl
from jax.experimental.pallas import tpu as pltpu
from jax.experimental.pallas import tpu_sc as plsc

_LANES = 16


@functools.partial(jax.jit, static_argnames=("nchunk", "k"))
def _pool(idx3, table, *, nchunk, k):
    B = idx3.shape[0]
    L = nchunk * k
    E = table.shape[1]

    info = plsc.get_sparse_core_info()
    nw = info.num_cores * info.num_subcores
    bpw = B // nw
    ncol = E // _LANES
    runroll = 4

    mesh = plsc.VectorSubcoreMesh(core_axis_name="c", subcore_axis_name="s")

    @functools.partial(
        pl.kernel,
        out_type=jax.ShapeDtypeStruct((B, E), jnp.float32),
        mesh=mesh,
        scratch_types=[
            pltpu.VMEM((bpw, nchunk * k), jnp.int32),
            pltpu.VMEM((L, E), jnp.float32),
            pltpu.VMEM((L, E), jnp.float32),
            pltpu.VMEM((bpw, E), jnp.float32),
            pltpu.SemaphoreType.DMA,
            pltpu.SemaphoreType.DMA,
        ],
        compiler_params=pltpu.CompilerParams(use_tc_tiling_on_sc=False),
    )
    def pool(texts_hbm, table_hbm, out_hbm, idx_v, rows0, rows1, pooled_v,
             sem0, sem1):
        wid = lax.axis_index("s") * info.num_cores + lax.axis_index("c")
        base = wid * bpw

        pltpu.sync_copy(texts_hbm.at[pl.ds(base, bpw)], idx_v)

        def gather_bag(b, buf, sem):
            for j in range(nchunk):
                pltpu.async_copy(
                    table_hbm.at[idx_v.at[b, pl.ds(j * k, k)]],
                    buf.at[pl.ds(j * k, k)],
                    sem,
                )

        def drain_bag(b, buf, sem):
            for j in range(nchunk):
                pltpu.make_async_copy(
                    table_hbm.at[idx_v.at[b, pl.ds(j * k, k)]],
                    buf.at[pl.ds(j * k, k)],
                    sem,
                ).wait()

        inv_l = jnp.float32(1.0 / L)

        def reduce_bag(b, buf):
            def rbody(r, accs):
                out = list(accs)
                for u in range(runroll):
                    row = r * runroll + u
                    for c in range(ncol):
                        out[c] = out[c] + buf[row, pl.ds(c * _LANES, _LANES)]
                return tuple(out)

            zero = jnp.zeros((_LANES,), jnp.float32)
            accs = lax.fori_loop(0, L // runroll, rbody, (zero,) * ncol)
            for c in range(ncol):
                pooled_v[b, pl.ds(c * _LANES, _LANES)] = accs[c] * inv_l

        gather_bag(0, rows0, sem0)

        def body(i, carry):
            b = 2 * i
            gather_bag(b + 1, rows1, sem1)
            drain_bag(b, rows0, sem0)
            reduce_bag(b, rows0)

            @pl.when(b + 2 < bpw)
            def _():
                gather_bag(b + 2, rows0, sem0)

            drain_bag(b + 1, rows1, sem1)
            reduce_bag(b + 1, rows1)
            return carry

        lax.fori_loop(0, bpw // 2, body, 0)

        pltpu.sync_copy(pooled_v, out_hbm.at[pl.ds(base, bpw)])

    return pool(idx3, table)


def _mlp(pooled, W1, b1, W2, b2):
    B, E = pooled.shape
    H = W1.shape[1]
    C = W2.shape[1]
    W2T = W2.T
    bb = 512
    cb = 1000
    grid = (B // bb, pl.cdiv(C, cb))

    def body(p_ref, w1_ref, b1_ref, w2t_ref, b2_ref, ot_ref):
        h = jnp.dot(p_ref[...], w1_ref[...],
                    preferred_element_type=jnp.float32) + b1_ref[...]
        h = jnp.maximum(h, 0.0)
        ot = jax.lax.dot_general(
            w2t_ref[...], h, (((1,), (1,)), ((), ())),
            preferred_element_type=jnp.float32)
        ot_ref[...] = ot + b2_ref[...]

    outT = pl.pallas_call(
        body,
        grid=grid,
        in_specs=[
            pl.BlockSpec((bb, E), lambda i, j: (i, 0)),
            pl.BlockSpec((E, H), lambda i, j: (0, 0)),
            pl.BlockSpec((1, H), lambda i, j: (0, 0)),
            pl.BlockSpec((cb, H), lambda i, j: (j, 0)),
            pl.BlockSpec((cb, 1), lambda i, j: (j, 0)),
        ],
        out_specs=pl.BlockSpec((cb, bb), lambda i, j: (j, i)),
        out_shape=jax.ShapeDtypeStruct((C, B), jnp.float32),
    )(pooled, W1, b1.reshape(1, H), W2T, b2.reshape(C, 1))
    return outT.T


def kernel(texts, table, W1, b1, W2, b2):
    B, L = texts.shape
    nchunk = 5
    k = L // nchunk
    pooled = _pool(texts.astype(jnp.int32), table, nchunk=nchunk, k=k)
    return _mlp(pooled, W1, b1, W2, b2)


# speedup vs baseline: 13.9216x; 1.0023x over previous
import functools

import jax
import jax.numpy as jnp
from jax import lax
from jax.experimental import pallas as pl
from jax.experimental.pallas import tpu as pltpu
from jax.experimental.pallas import tpu_sc as plsc

_LANES = 16


@functools.partial(jax.jit, static_argnames=("nchunk", "k"))
def _pool(idx3, table, *, nchunk, k):
    B = idx3.shape[0]
    L = nchunk * k
    E = table.shape[1]

    info = plsc.get_sparse_core_info()
    nw = info.num_cores * info.num_subcores
    bpw = B // nw
    ncol = E // _LANES
    runroll = 4

    mesh = plsc.VectorSubcoreMesh(core_axis_name="c", subcore_axis_name="s")

    @functools.partial(
        pl.kernel,
        out_type=jax.ShapeDtypeStruct((B, E), jnp.float32),
        mesh=mesh,
        scratch_types=[
            pltpu.VMEM((bpw, nchunk * k), jnp.int32),
            pltpu.VMEM((L, E), jnp.float32),
            pltpu.VMEM((L, E), jnp.float32),
            pltpu.VMEM((bpw, E), jnp.float32),
            pltpu.SemaphoreType.DMA,
            pltpu.SemaphoreType.DMA,
        ],
        compiler_params=pltpu.CompilerParams(use_tc_tiling_on_sc=False),
    )
    def pool(texts_hbm, table_hbm, out_hbm, idx_v, rows0, rows1, pooled_v,
             sem0, sem1):
        wid = lax.axis_index("s") * info.num_cores + lax.axis_index("c")
        base = wid * bpw

        pltpu.sync_copy(texts_hbm.at[pl.ds(base, bpw)], idx_v)

        def gather_bag(b, buf, sem):
            for j in range(nchunk):
                pltpu.async_copy(
                    table_hbm.at[idx_v.at[b, pl.ds(j * k, k)]],
                    buf.at[pl.ds(j * k, k)],
                    sem,
                )

        def drain_bag(b, buf, sem):
            for j in range(nchunk):
                pltpu.make_async_copy(
                    table_hbm.at[idx_v.at[b, pl.ds(j * k, k)]],
                    buf.at[pl.ds(j * k, k)],
                    sem,
                ).wait()

        inv_l = jnp.float32(1.0 / L)

        def reduce_bag(b, buf):
            def rbody(r, accs):
                out = list(accs)
                for u in range(runroll):
                    row = r * runroll + u
                    for c in range(ncol):
                        out[c] = out[c] + buf[row, pl.ds(c * _LANES, _LANES)]
                return tuple(out)

            zero = jnp.zeros((_LANES,), jnp.float32)
            accs = lax.fori_loop(0, L // runroll, rbody, (zero,) * ncol)
            for c in range(ncol):
                pooled_v[b, pl.ds(c * _LANES, _LANES)] = accs[c] * inv_l

        gather_bag(0, rows0, sem0)

        def body(i, carry):
            b = 2 * i
            gather_bag(b + 1, rows1, sem1)
            drain_bag(b, rows0, sem0)
            reduce_bag(b, rows0)

            @pl.when(b + 2 < bpw)
            def _():
                gather_bag(b + 2, rows0, sem0)

            drain_bag(b + 1, rows1, sem1)
            reduce_bag(b + 1, rows1)
            return carry

        lax.fori_loop(0, bpw // 2, body, 0)

        pltpu.sync_copy(pooled_v, out_hbm.at[pl.ds(base, bpw)])

    return pool(idx3, table)


def _mlp(pooled, W1, b1, W2, b2):
    B, E = pooled.shape
    H = W1.shape[1]
    C = W2.shape[1]
    W1T = W1.T
    W2T = W2.T
    bb = 512
    cb = 1000
    grid = (B // bb, pl.cdiv(C, cb))

    def body(p_ref, w1t_ref, b1_ref, w2t_ref, b2_ref, ot_ref):
        ht = jax.lax.dot_general(
            w1t_ref[...], p_ref[...], (((1,), (1,)), ((), ())),
            preferred_element_type=jnp.float32) + b1_ref[...]
        ht = jnp.maximum(ht, 0.0)
        ot = jnp.dot(w2t_ref[...], ht, preferred_element_type=jnp.float32)
        ot_ref[...] = ot + b2_ref[...]

    outT = pl.pallas_call(
        body,
        grid=grid,
        in_specs=[
            pl.BlockSpec((bb, E), lambda i, j: (i, 0)),
            pl.BlockSpec((H, E), lambda i, j: (0, 0)),
            pl.BlockSpec((H, 1), lambda i, j: (0, 0)),
            pl.BlockSpec((cb, H), lambda i, j: (j, 0)),
            pl.BlockSpec((cb, 1), lambda i, j: (j, 0)),
        ],
        out_specs=pl.BlockSpec((cb, bb), lambda i, j: (j, i)),
        out_shape=jax.ShapeDtypeStruct((C, B), jnp.float32),
    )(pooled, W1T, b1.reshape(H, 1), W2T, b2.reshape(C, 1))
    return outT.T


def kernel(texts, table, W1, b1, W2, b2):
    B, L = texts.shape
    V, E = table.shape
    nchunk = 5
    k = L // nchunk
    pooled = _pool(texts.astype(jnp.int32), table, nchunk=nchunk, k=k)
    return _mlp(pooled, W1, b1, W2, b2)


# speedup vs baseline: 18.2984x; 1.3144x over previous
import functools

import jax
import jax.numpy as jnp
from jax import lax
from jax.experimental import pallas as pl
from jax.experimental.pallas import tpu as pltpu
from jax.experimental.pallas import tpu_sc as plsc

_LANES = 16


@functools.partial(jax.jit, static_argnames=("nchunk", "k"))
def _pool(idx3, table, *, nchunk, k):
    B = idx3.shape[0]
    L = nchunk * k
    E = table.shape[1]

    info = plsc.get_sparse_core_info()
    nw = info.num_cores * info.num_subcores
    bpw = B // nw
    ncol = E // _LANES
    runroll = 8

    mesh = plsc.VectorSubcoreMesh(core_axis_name="c", subcore_axis_name="s")

    nbuf = 4

    @functools.partial(
        pl.kernel,
        out_type=jax.ShapeDtypeStruct((B, E), jnp.float32),
        mesh=mesh,
        scratch_types=[
            pltpu.VMEM((bpw, nchunk * k), jnp.int32),
            pltpu.VMEM((nbuf, L, E), jnp.float32),
            pltpu.VMEM((bpw, E), jnp.float32),
        ] + [pltpu.SemaphoreType.DMA] * nbuf,
        compiler_params=pltpu.CompilerParams(use_tc_tiling_on_sc=False),
    )
    def pool(texts_hbm, table_hbm, out_hbm, idx_v, rows_v, pooled_v, *sems):
        wid = lax.axis_index("s") * info.num_cores + lax.axis_index("c")
        base = wid * bpw

        pltpu.sync_copy(texts_hbm.at[pl.ds(base, bpw)], idx_v)

        def gather_bag(b, u):
            for j in range(nchunk):
                pltpu.async_copy(
                    table_hbm.at[idx_v.at[b, pl.ds(j * k, k)]],
                    rows_v.at[u, pl.ds(j * k, k)],
                    sems[u],
                )

        def drain_bag(b, u):
            for j in range(nchunk):
                pltpu.make_async_copy(
                    table_hbm.at[idx_v.at[b, pl.ds(j * k, k)]],
                    rows_v.at[u, pl.ds(j * k, k)],
                    sems[u],
                ).wait()

        inv_l = jnp.float32(1.0 / L)

        def reduce_bag(b, u):
            def rbody(r, accs):
                out = list(accs)
                for v in range(runroll):
                    row = r * runroll + v
                    for c in range(ncol):
                        out[c] = out[c] + rows_v[u, row,
                                                 pl.ds(c * _LANES, _LANES)]
                return tuple(out)

            zero = jnp.zeros((_LANES,), jnp.float32)
            accs = lax.fori_loop(0, L // runroll, rbody, (zero,) * ncol)
            for c in range(ncol):
                pooled_v[b, pl.ds(c * _LANES, _LANES)] = accs[c] * inv_l

        for u in range(nbuf - 1):
            gather_bag(u, u)

        def body(q, carry):
            bq = nbuf * q
            for u in range(nbuf):
                b = bq + u
                nxt = b + nbuf - 1

                @pl.when(nxt < bpw)
                def _():
                    gather_bag(nxt, (u + nbuf - 1) % nbuf)

                drain_bag(b, u)
                reduce_bag(b, u)
            return carry

        lax.fori_loop(0, bpw // nbuf, body, 0)

        pltpu.sync_copy(pooled_v, out_hbm.at[pl.ds(base, bpw)])

    return pool(idx3, table)


def _mlp(pooled, W1, b1, W2, b2):
    B, E = pooled.shape
    H = W1.shape[1]
    C = W2.shape[1]
    W1T = W1.T
    W2T = W2.T
    cb = 600
    grid = (pl.cdiv(C, cb),)

    def body(p_ref, w1t_ref, b1_ref, w2t_ref, b2_ref, ot_ref):
        ht = jax.lax.dot_general(
            w1t_ref[...], p_ref[...], (((1,), (1,)), ((), ())),
            preferred_element_type=jnp.float32) + b1_ref[...]
        ht = jnp.maximum(ht, 0.0)
        ot = jnp.dot(w2t_ref[...], ht, preferred_element_type=jnp.float32)
        ot_ref[...] = ot + b2_ref[...]

    outT = pl.pallas_call(
        body,
        grid=grid,
        in_specs=[
            pl.BlockSpec((B, E), lambda j: (0, 0)),
            pl.BlockSpec((H, E), lambda j: (0, 0)),
            pl.BlockSpec((H, 1), lambda j: (0, 0)),
            pl.BlockSpec((cb, H), lambda j: (j, 0)),
            pl.BlockSpec((cb, 1), lambda j: (j, 0)),
        ],
        out_specs=pl.BlockSpec((cb, B), lambda j: (j, 0)),
        out_shape=jax.ShapeDtypeStruct((C, B), jnp.float32),
    )(pooled, W1T, b1.reshape(H, 1), W2T, b2.reshape(C, 1))
    return outT.T


def kernel(texts, table, W1, b1, W2, b2):
    B, L = texts.shape
    V, E = table.shape
    nchunk = 5
    k = L // nchunk
    pooled = _pool(texts.astype(jnp.int32), table, nchunk=nchunk, k=k)
    return _mlp(pooled, W1, b1, W2, b2)
